# baseline (speedup 1.0000x reference)
import jax
import jax.numpy as jnp
import numpy as np
from jax.experimental import pallas as pl
from jax.experimental.pallas import tpu as pltpu

_N = 64
_F = 64
_ND = _N * _F
_E_BLK = 512

_C11 = (((1,), (1,)), ((), ()))
_C00 = (((0,), (0,)), ((), ()))


def _lane_tree(v, op):
    w = v.shape[1]
    while w > _F:
        h = w // 2
        v = op(v[:, :h], v[:, h:w])
        w = h
    return v


def _split_dot(a, sel, dims):
    bf16 = jnp.bfloat16
    f32 = jnp.float32
    hi = a.astype(bf16)
    lo = (a - hi.astype(f32)).astype(bf16)
    dg = jax.lax.dot_general
    return (dg(hi, sel, dims, preferred_element_type=f32)
            + dg(lo, sel, dims, preferred_element_type=f32))


def _cheshire_kernel(feat, we, inc, ber, gnw, gnb, gms,
                     cheb, chebb, wlin, blin, knb4, kpb, out):
    f32 = jnp.float32
    dg = jax.lax.dot_general
    minc = (inc[...] != 0).astype(f32)
    c = dg(minc, jnp.ones((_N, 1), f32), _C00,
           preferred_element_type=f32)

    x = jnp.clip(dg(feat[...], we[...], _C11, preferred_element_type=f32)
                 + ber[...], -1.0, 1.0)

    mx = dg(minc, x, _C00, preferred_element_type=f32)
    mx2 = dg(minc, x * x, _C00, preferred_element_type=f32)
    inv_c = 1.0 / c
    mean = mx * inv_c
    mean_s = mean * gms[...]
    var = mx2 * inv_c - (2.0 * mean_s) * mean + mean_s * mean_s
    g = gnw[...] * jax.lax.rsqrt(var + 1e-5)
    h = gnb[...] - mean_s * g
    s1 = mx * g + c * h

    a2 = jnp.where(c > 1.0, 1.0 / (c - 1.0), 0.0)
    a4 = a2 * a2
    beta = 2.0 * a4 - 1.0

    w0 = cheb[0]
    w1 = cheb[1]
    w2 = cheb[2]

    per = (dg(h, w0, _C11, preferred_element_type=f32)
           + a2 * dg(h, w1, _C11, preferred_element_type=f32)
           + beta * dg(h, w2, _C11, preferred_element_type=f32)
           - a2 * dg(s1, w1, _C11, preferred_element_type=f32)
           - (2.0 * a4 * (2.0 - c)) * dg(s1, w2, _C11, preferred_element_type=f32)
           + chebb[...])

    xrep4 = _split_dot(x, knb4[...], _C00)
    wcat = jnp.concatenate([w0, w1, w2], axis=1) * 0.25
    wtcat = _split_dot(wcat, kpb[...], _C00)
    kp32 = kpb[...].astype(f32)
    tcat = jnp.concatenate(
        [xrep4 * wtcat[:_F], xrep4 * wtcat[_F:2 * _F],
         xrep4 * wtcat[2 * _F:], kp32], axis=0)
    gcat = jnp.concatenate([g, a2 * g, beta * g, per], axis=1)

    q = jnp.clip(jnp.dot(gcat, tcat, preferred_element_type=f32), -1.0, 1.0)
    maskB = dg(minc.astype(jnp.bfloat16), knb4[...], _C00,
               preferred_element_type=f32)
    s4 = q * maskB
    ymax = 0.25 * _lane_tree(s4 + maskB, jnp.maximum) - 1.0
    ymin = 0.25 * _lane_tree(s4 - maskB, jnp.minimum) + 1.0
    ynorm = 0.25 * jnp.sqrt(_lane_tree(s4 * s4, jnp.add) * inv_c)
    y = jnp.concatenate([ymax - ymin, ynorm], axis=1)
    z = jnp.sum(y * wlin[...], axis=1, keepdims=True) + blin[...]
    out[...] = jax.nn.sigmoid(z)


@jax.jit
def kernel(feature, incidence_matrix, W_enc, b_enc, gn_weight, gn_bias,
           gn_mean_scale, cheb_W, cheb_b, W_lin, b_lin):
    f32 = jnp.float32
    bf16 = jnp.bfloat16
    feat = feature.astype(f32)
    we = W_enc.astype(f32)
    n_he = incidence_matrix.shape[1]
    n = feat.shape[0]
    in_dim = feat.shape[1]
    emb = we.shape[0]
    conv = cheb_W.shape[1]

    knb4 = jnp.asarray(4.0 * np.kron(np.eye(n), np.ones((1, conv))),
                       dtype=bf16)
    kpb = jnp.asarray(np.tile(np.eye(conv), (1, n)), dtype=bf16)

    args = (
        feat,
        we,
        incidence_matrix,
        b_enc.reshape(1, emb).astype(f32),
        gn_weight.reshape(1, emb).astype(f32),
        gn_bias.reshape(1, emb).astype(f32),
        gn_mean_scale.reshape(1, emb).astype(f32),
        cheb_W.astype(f32),
        cheb_b.reshape(1, conv).astype(f32),
        W_lin.reshape(1, 2 * conv).astype(f32),
        b_lin.reshape(1, 1).astype(f32),
        knb4,
        kpb,
    )

    def full(shape):
        return pl.BlockSpec(shape, lambda *_: tuple(0 for _ in shape))

    in_specs = [
        full((n, in_dim)),
        full((emb, in_dim)),
        pl.BlockSpec((n, _E_BLK), lambda i: (0, i)),
        full((1, emb)),
        full((1, emb)),
        full((1, emb)),
        full((1, emb)),
        full((3, conv, emb)),
        full((1, conv)),
        full((1, 2 * conv)),
        full((1, 1)),
        full((n, n * conv)),
        full((conv, n * conv)),
    ]

    return pl.pallas_call(
        _cheshire_kernel,
        grid=(n_he // _E_BLK,),
        in_specs=in_specs,
        out_specs=pl.BlockSpec((_E_BLK, 1), lambda i: (i, 0)),
        out_shape=jax.ShapeDtypeStruct((n_he, 1), f32),
        compiler_params=pltpu.CompilerParams(
            dimension_semantics=("parallel",)),
    )(*args)


# speedup vs baseline: 1.0847x; 1.0847x over previous
import jax
import jax.numpy as jnp
import numpy as np
from jax.experimental import pallas as pl
from jax.experimental.pallas import tpu as pltpu

_N = 64
_F = 64
_ND = _N * _F
_E_BLK = 1024

_C11 = (((1,), (1,)), ((), ()))
_C00 = (((0,), (0,)), ((), ()))


def _lane_tree(v, op):
    w = v.shape[1]
    while w > _F:
        h = w // 2
        v = op(v[:, :h], v[:, h:w])
        w = h
    return v


def _split_dot(a, sel, dims):
    bf16 = jnp.bfloat16
    f32 = jnp.float32
    hi = a.astype(bf16)
    lo = (a - hi.astype(f32)).astype(bf16)
    dg = jax.lax.dot_general
    return (dg(hi, sel, dims, preferred_element_type=f32)
            + dg(lo, sel, dims, preferred_element_type=f32))


def _cheshire_kernel(feat, we, inc, ber, gnw, gnb, gms,
                     cheb, chebb, wlin, blin, knb4, kpb, out):
    f32 = jnp.float32
    dg = jax.lax.dot_general
    minc = (inc[...] != 0).astype(f32)
    c = dg(minc, jnp.ones((_N, 1), f32), _C00,
           preferred_element_type=f32)

    x = jnp.clip(dg(feat[...], we[...], _C11, preferred_element_type=f32)
                 + ber[...], -1.0, 1.0)

    mx = dg(minc, x, _C00, preferred_element_type=f32)
    mx2 = dg(minc, x * x, _C00, preferred_element_type=f32)
    inv_c = 1.0 / c
    mean = mx * inv_c
    mean_s = mean * gms[...]
    var = mx2 * inv_c - (2.0 * mean_s) * mean + mean_s * mean_s
    g = gnw[...] * jax.lax.rsqrt(var + 1e-5)
    h = gnb[...] - mean_s * g
    s1 = mx * g + c * h

    a2 = jnp.where(c > 1.0, 1.0 / (c - 1.0), 0.0)
    a4 = a2 * a2
    beta = 2.0 * a4 - 1.0

    w0 = cheb[0]
    w1 = cheb[1]
    w2 = cheb[2]

    per = (dg(h, w0, _C11, preferred_element_type=f32)
           + a2 * dg(h, w1, _C11, preferred_element_type=f32)
           + beta * dg(h, w2, _C11, preferred_element_type=f32)
           - a2 * dg(s1, w1, _C11, preferred_element_type=f32)
           - (2.0 * a4 * (2.0 - c)) * dg(s1, w2, _C11, preferred_element_type=f32)
           + chebb[...])

    xrep4 = _split_dot(x, knb4[...], _C00)
    wcat = jnp.concatenate([w0, w1, w2], axis=1) * 0.25
    wtcat = _split_dot(wcat, kpb[...], _C00)
    kp32 = kpb[...].astype(f32)
    tcat = jnp.concatenate(
        [xrep4 * wtcat[:_F], xrep4 * wtcat[_F:2 * _F],
         xrep4 * wtcat[2 * _F:], kp32], axis=0)
    gcat = jnp.concatenate([g, a2 * g, beta * g, per], axis=1)

    q = jnp.clip(jnp.dot(gcat, tcat, preferred_element_type=f32), -1.0, 1.0)
    maskB = dg(minc.astype(jnp.bfloat16), knb4[...], _C00,
               preferred_element_type=f32)
    s4 = q * maskB
    ymax = 0.25 * _lane_tree(s4 + maskB, jnp.maximum) - 1.0
    ymin = 0.25 * _lane_tree(s4 - maskB, jnp.minimum) + 1.0
    ynorm = 0.25 * jnp.sqrt(_lane_tree(s4 * s4, jnp.add) * inv_c)
    y = jnp.concatenate([ymax - ymin, ynorm], axis=1)
    z = jnp.sum(y * wlin[...], axis=1, keepdims=True) + blin[...]
    out[...] = jax.nn.sigmoid(z)


@jax.jit
def kernel(feature, incidence_matrix, W_enc, b_enc, gn_weight, gn_bias,
           gn_mean_scale, cheb_W, cheb_b, W_lin, b_lin):
    f32 = jnp.float32
    bf16 = jnp.bfloat16
    feat = feature.astype(f32)
    we = W_enc.astype(f32)
    n_he = incidence_matrix.shape[1]
    n = feat.shape[0]
    in_dim = feat.shape[1]
    emb = we.shape[0]
    conv = cheb_W.shape[1]

    knb4 = jnp.asarray(4.0 * np.kron(np.eye(n), np.ones((1, conv))),
                       dtype=bf16)
    kpb = jnp.asarray(np.tile(np.eye(conv), (1, n)), dtype=bf16)

    args = (
        feat,
        we,
        incidence_matrix,
        b_enc.reshape(1, emb).astype(f32),
        gn_weight.reshape(1, emb).astype(f32),
        gn_bias.reshape(1, emb).astype(f32),
        gn_mean_scale.reshape(1, emb).astype(f32),
        cheb_W.astype(f32),
        cheb_b.reshape(1, conv).astype(f32),
        W_lin.reshape(1, 2 * conv).astype(f32),
        b_lin.reshape(1, 1).astype(f32),
        knb4,
        kpb,
    )

    def full(shape):
        return pl.BlockSpec(shape, lambda *_: tuple(0 for _ in shape))

    in_specs = [
        full((n, in_dim)),
        full((emb, in_dim)),
        pl.BlockSpec((n, _E_BLK), lambda i: (0, i)),
        full((1, emb)),
        full((1, emb)),
        full((1, emb)),
        full((1, emb)),
        full((3, conv, emb)),
        full((1, conv)),
        full((1, 2 * conv)),
        full((1, 1)),
        full((n, n * conv)),
        full((conv, n * conv)),
    ]

    return pl.pallas_call(
        _cheshire_kernel,
        grid=(n_he // _E_BLK,),
        in_specs=in_specs,
        out_specs=pl.BlockSpec((_E_BLK, 1), lambda i: (i, 0)),
        out_shape=jax.ShapeDtypeStruct((n_he, 1), f32),
        compiler_params=pltpu.CompilerParams(
            dimension_semantics=("parallel",)),
    )(*args)
